# baseline (speedup 1.0000x reference)
import functools
import numpy as np

import jax
import jax.numpy as jnp
from jax import lax
from jax.experimental import pallas as pl
from jax.experimental.pallas import tpu as pltpu

_ROT1 = (13, 15, 26, 6)
_ROT2 = (17, 29, 16, 24)
_KS0 = np.int32(0)
_KS1 = np.int32(42)
_KS2 = np.int32(np.uint32(0) ^ np.uint32(42) ^ np.uint32(0x1BD11BDA))
_TINY = np.float32(np.finfo(np.float32).tiny)
_ONE_BITS = np.int32(0x3F800000)
_NEG_INF = np.float32(-np.inf)
_BIG_IDX = np.int32(2**31 - 1)


def _rotl(x, r):
    return lax.shift_left(x, np.int32(r)) | lax.shift_right_logical(
        x, np.int32(32 - r))


def _rounds(x0, x1, rots):
    for r in rots:
        x0 = x0 + x1
        x1 = _rotl(x1, r) ^ x0
    return x0, x1


def _threefry_bits(n):
    x1 = n + _KS1
    x0 = x1
    x1 = _rotl(x1, _ROT1[0]) ^ x0
    for r in _ROT1[1:]:
        x0 = x0 + x1
        x1 = _rotl(x1, r) ^ x0
    x0 = x0 + _KS1
    x1 = x1 + np.int32(_KS2 + np.int32(1))
    x0, x1 = _rounds(x0, x1, _ROT2)
    x0 = x0 + _KS2
    x1 = x1 + np.int32(2)
    x0, x1 = _rounds(x0, x1, _ROT1)
    x1 = x1 + np.int32(_KS1 + np.int32(3))
    x0, x1 = _rounds(x0, x1, _ROT2)
    x0 = x0 + _KS1
    x1 = x1 + np.int32(_KS2 + np.int32(4))
    x0, x1 = _rounds(x0, x1, _ROT1)
    x0 = x0 + _KS2
    x1 = x1 + np.int32(5)
    return x0 ^ x1


def _sample_kernel(logits_ref, val_out, idx_out, best_val, best_idx, *, blk,
                   cols, nblk_half):
    c = pl.program_id(0)
    i = pl.program_id(1)

    @pl.when(i == 0)
    def _init():
        best_val[...] = jnp.full_like(best_val, _NEG_INF)
        best_idx[...] = jnp.zeros_like(best_idx)

    rows = logits_ref.shape[0]
    nblk_tot = (cols + blk - 1) // blk
    base = jnp.minimum(c * nblk_half + i, nblk_tot - 1) * blk
    col = base + lax.broadcasted_iota(jnp.int32, (rows, blk), 1)
    row = lax.broadcasted_iota(jnp.int32, (rows, blk), 0)
    n = row * np.int32(cols) + col

    bits = _threefry_bits(n)
    fbits = lax.shift_right_logical(bits, np.int32(9)) | _ONE_BITS
    u = lax.bitcast_convert_type(fbits, jnp.float32) - np.float32(1.0) + _TINY
    g = -jnp.log(-jnp.log(u))
    val = g + logits_ref[...]
    val = jnp.where(col < np.int32(cols), val, _NEG_INF)

    bmax = jnp.max(val, axis=1, keepdims=True)
    bidx = jnp.min(jnp.where(val == bmax, col, _BIG_IDX), axis=1,
                   keepdims=True)

    upd = bmax > best_val[...]
    best_val[...] = jnp.where(upd, bmax, best_val[...])
    best_idx[...] = jnp.where(upd, bidx, best_idx[...])

    @pl.when(i == nblk_half - 1)
    def _emit():
        val_out[...] = best_val[...]
        idx_out[...] = best_idx[...]


def _merge_kernel(val_ref, idx_ref, out_ref):
    nparts = val_ref.shape[0]
    best_v = val_ref[0]
    best_i = idx_ref[0]
    for p in range(1, nparts):
        take = val_ref[p] > best_v
        best_v = jnp.where(take, val_ref[p], best_v)
        best_i = jnp.where(take, idx_ref[p], best_i)
    out_ref[...] = best_i


def kernel(logits):
    rows, cols = logits.shape
    blk = 4096
    nparts = 2
    nblk_half = (cols + nparts * blk - 1) // (nparts * blk)
    val, idx = pl.pallas_call(
        functools.partial(_sample_kernel, blk=blk, cols=cols,
                          nblk_half=nblk_half),
        grid=(nparts, nblk_half),
        in_specs=[pl.BlockSpec(
            (rows, blk),
            lambda c, i, nbh=nblk_half, nbt=(cols + blk - 1) // blk:
                (0, jnp.minimum(c * nbh + i, nbt - 1)))],
        out_specs=[
            pl.BlockSpec((1, rows, 1), lambda c, i: (c, 0, 0)),
            pl.BlockSpec((1, rows, 1), lambda c, i: (c, 0, 0)),
        ],
        out_shape=[
            jax.ShapeDtypeStruct((nparts, rows, 1), jnp.float32),
            jax.ShapeDtypeStruct((nparts, rows, 1), jnp.int32),
        ],
        scratch_shapes=[
            pltpu.VMEM((1, rows, 1), jnp.float32),
            pltpu.VMEM((1, rows, 1), jnp.int32),
        ],
        compiler_params=pltpu.CompilerParams(
            dimension_semantics=("parallel", "arbitrary")),
    )(logits)
    out = pl.pallas_call(
        _merge_kernel,
        out_shape=jax.ShapeDtypeStruct((rows, 1), jnp.int32),
    )(val, idx)
    return out.reshape(-1)


# speedup vs baseline: 1.0112x; 1.0112x over previous
import functools
import numpy as np

import jax
import jax.numpy as jnp
from jax import lax
from jax.experimental import pallas as pl
from jax.experimental.pallas import tpu as pltpu

_ROT1 = (13, 15, 26, 6)
_ROT2 = (17, 29, 16, 24)
_KS1 = np.int32(42)
_KS2 = np.int32(np.uint32(0) ^ np.uint32(42) ^ np.uint32(0x1BD11BDA))
_TINY = np.float32(np.finfo(np.float32).tiny)
_ONE_BITS = np.int32(0x3F800000)
_NEG_INF = np.float32(-np.inf)
_BIG_IDX = np.int32(2**31 - 1)


def _rotl(x, r):
    return lax.shift_left(x, np.int32(r)) | lax.shift_right_logical(
        x, np.int32(32 - r))


def _rounds(x0, x1, rots):
    for r in rots:
        x0 = x0 + x1
        x1 = _rotl(x1, r) ^ x0
    return x0, x1


def _threefry_bits(x1):
    x0 = x1
    x1 = _rotl(x1, _ROT1[0]) ^ x0
    for r in _ROT1[1:]:
        x0 = x0 + x1
        x1 = _rotl(x1, r) ^ x0
    x0 = x0 + _KS1
    x1 = x1 + np.int32(_KS2 + np.int32(1))
    x0, x1 = _rounds(x0, x1, _ROT2)
    x0 = x0 + _KS2
    x1 = x1 + np.int32(2)
    x0, x1 = _rounds(x0, x1, _ROT1)
    x1 = x1 + np.int32(_KS1 + np.int32(3))
    x0, x1 = _rounds(x0, x1, _ROT2)
    x0 = x0 + _KS1
    x1 = x1 + np.int32(_KS2 + np.int32(4))
    x0, x1 = _rounds(x0, x1, _ROT1)
    x0 = x0 + _KS2
    x1 = x1 + np.int32(5)
    return x0 ^ x1


def _sample_kernel(logits_ref, out_ref, best_val, best_idx, seed_ref, *, blk,
                   cols, nblk):
    i = pl.program_id(0)
    rows = logits_ref.shape[0]

    @pl.when(i == 0)
    def _init():
        best_val[...] = jnp.full_like(best_val, _NEG_INF)
        best_idx[...] = jnp.zeros_like(best_idx)
        col0 = lax.broadcasted_iota(jnp.int32, (rows, blk), 1)
        row = lax.broadcasted_iota(jnp.int32, (rows, blk), 0)
        seed_ref[...] = row * np.int32(cols) + col0 + _KS1

    base = i * blk
    x1 = seed_ref[...] + base
    bits = _threefry_bits(x1)
    fbits = lax.shift_right_logical(bits, np.int32(9)) | _ONE_BITS
    u = lax.bitcast_convert_type(fbits, jnp.float32) - np.float32(1.0) + _TINY
    g = -jnp.log(-jnp.log(u))
    val = g + logits_ref[...]
    ci = lax.broadcasted_iota(jnp.int32, (rows, blk), 1)
    val = jnp.where(ci < np.int32(cols) - base, val, _NEG_INF)

    bmax = jnp.max(val, axis=1, keepdims=True)
    bidx = base + jnp.min(jnp.where(val == bmax, ci, _BIG_IDX), axis=1,
                          keepdims=True)
    upd = bmax > best_val[...]
    best_val[...] = jnp.where(upd, bmax, best_val[...])
    best_idx[...] = jnp.where(upd, bidx, best_idx[...])

    @pl.when(i == nblk - 1)
    def _emit():
        out_ref[...] = best_idx[...]


def kernel(logits):
    rows, cols = logits.shape
    blk = 4096
    nblk = (cols + blk - 1) // blk
    out = pl.pallas_call(
        functools.partial(_sample_kernel, blk=blk, cols=cols, nblk=nblk),
        grid=(nblk,),
        in_specs=[pl.BlockSpec((rows, blk), lambda i: (0, i))],
        out_specs=pl.BlockSpec((rows, 1), lambda i: (0, 0)),
        out_shape=jax.ShapeDtypeStruct((rows, 1), jnp.int32),
        scratch_shapes=[
            pltpu.VMEM((rows, 1), jnp.float32),
            pltpu.VMEM((rows, 1), jnp.int32),
            pltpu.VMEM((rows, blk), jnp.int32),
        ],
    )(logits)
    return out.reshape(-1)


# speedup vs baseline: 1.0896x; 1.0776x over previous
import functools
import numpy as np

import jax
import jax.numpy as jnp
from jax import lax
from jax.experimental import pallas as pl
from jax.experimental.pallas import tpu as pltpu

_ROT1 = (13, 15, 26, 6)
_ROT2 = (17, 29, 16, 24)
_KS1 = np.int32(42)
_KS2 = np.int32(np.uint32(0) ^ np.uint32(42) ^ np.uint32(0x1BD11BDA))
_TINY = np.float32(np.finfo(np.float32).tiny)
_ONE_BITS = np.int32(0x3F800000)
_NEG_INF = np.float32(-np.inf)
_NEG_LN2 = np.float32(-np.log(2.0))
_BIG_IDX = np.int32(2**31 - 1)


def _rotl(x, r):
    return lax.shift_left(x, np.int32(r)) | lax.shift_right_logical(
        x, np.int32(32 - r))


def _rounds(x0, x1, rots):
    for r in rots:
        x0 = x0 + x1
        x1 = _rotl(x1, r) ^ x0
    return x0, x1


def _threefry_bits(x1):
    x0 = x1
    x1 = _rotl(x1, _ROT1[0]) ^ x0
    for r in _ROT1[1:]:
        x0 = x0 + x1
        x1 = _rotl(x1, r) ^ x0
    x0 = x0 + _KS1
    x1 = x1 + np.int32(_KS2 + np.int32(1))
    x0, x1 = _rounds(x0, x1, _ROT2)
    x0 = x0 + _KS2
    x1 = x1 + np.int32(2)
    x0, x1 = _rounds(x0, x1, _ROT1)
    x1 = x1 + np.int32(_KS1 + np.int32(3))
    x0, x1 = _rounds(x0, x1, _ROT2)
    x0 = x0 + _KS1
    x1 = x1 + np.int32(_KS2 + np.int32(4))
    x0, x1 = _rounds(x0, x1, _ROT1)
    x0 = x0 + _KS2
    x1 = x1 + np.int32(5)
    return x0 ^ x1


def _sample_kernel(logits_ref, out_ref, best_val, best_idx, seed_ref, *, blk,
                   cols, nblk):
    i = pl.program_id(0)
    rows = logits_ref.shape[0]

    @pl.when(i == 0)
    def _init():
        best_val[...] = jnp.full_like(best_val, _NEG_INF)
        best_idx[...] = jnp.zeros_like(best_idx)
        col0 = lax.broadcasted_iota(jnp.int32, (rows, blk), 1)
        row = lax.broadcasted_iota(jnp.int32, (rows, blk), 0)
        seed_ref[...] = row * np.int32(cols) + col0 + _KS1

    base = i * blk
    x1 = seed_ref[...] + base
    bits = _threefry_bits(x1)
    fbits = lax.shift_right_logical(bits, np.int32(9)) | _ONE_BITS
    u = lax.bitcast_convert_type(fbits, jnp.float32) - np.float32(1.0) + _TINY
    g = -jnp.log(-jnp.log(u))
    val = g + logits_ref[...]
    ci = lax.broadcasted_iota(jnp.int32, (rows, blk), 1)
    val = jnp.where(ci < np.int32(cols) - base, val, _NEG_INF)

    bmax = jnp.max(val, axis=1, keepdims=True)
    bidx = base + jnp.min(jnp.where(val == bmax, ci, _BIG_IDX), axis=1,
                          keepdims=True)
    upd = bmax > best_val[...]
    best_val[...] = jnp.where(upd, bmax, best_val[...])
    best_idx[...] = jnp.where(upd, bidx, best_idx[...])

    @pl.when(i == nblk - 1)
    def _emit():
        out_ref[...] = best_idx[...]


def kernel(logits):
    rows, cols = logits.shape
    blk = 2048
    nblk = (cols + blk - 1) // blk
    out = pl.pallas_call(
        functools.partial(_sample_kernel, blk=blk, cols=cols, nblk=nblk),
        grid=(nblk,),
        in_specs=[pl.BlockSpec((rows, blk), lambda i: (0, i))],
        out_specs=pl.BlockSpec((rows, 1), lambda i: (0, 0)),
        out_shape=jax.ShapeDtypeStruct((rows, 1), jnp.int32),
        scratch_shapes=[
            pltpu.VMEM((rows, 1), jnp.float32),
            pltpu.VMEM((rows, 1), jnp.int32),
            pltpu.VMEM((rows, blk), jnp.int32),
        ],
    )(logits)
    return out.reshape(-1)


# speedup vs baseline: 1.1257x; 1.0331x over previous
import functools
import numpy as np

import jax
import jax.numpy as jnp
from jax import lax
from jax.experimental import pallas as pl
from jax.experimental.pallas import tpu as pltpu
from jax.experimental.pallas import tpu_sc as plsc

_ROT1 = (13, 15, 26, 6)
_ROT2 = (17, 29, 16, 24)
_KS1 = np.int32(42)
_KS2 = np.int32(np.uint32(0) ^ np.uint32(42) ^ np.uint32(0x1BD11BDA))
_TINY = np.float32(np.finfo(np.float32).tiny)
_ONE_BITS = np.int32(0x3F800000)
_NEG_INF = np.float32(-np.inf)
_BIG_IDX = np.int32(2**31 - 1)

_BLK = 2048
_SC_CHUNK = 2048
_SC_ROWS_PER_SUBCORE = 2


def _rotl(x, r):
    return lax.shift_left(x, np.int32(r)) | lax.shift_right_logical(
        x, np.int32(32 - r))


def _rounds(x0, x1, rots):
    for r in rots:
        x0 = x0 + x1
        x1 = _rotl(x1, r) ^ x0
    return x0, x1


def _threefry_bits(x1):
    x0 = x1
    x1 = _rotl(x1, _ROT1[0]) ^ x0
    for r in _ROT1[1:]:
        x0 = x0 + x1
        x1 = _rotl(x1, r) ^ x0
    x0 = x0 + _KS1
    x1 = x1 + np.int32(_KS2 + np.int32(1))
    x0, x1 = _rounds(x0, x1, _ROT2)
    x0 = x0 + _KS2
    x1 = x1 + np.int32(2)
    x0, x1 = _rounds(x0, x1, _ROT1)
    x1 = x1 + np.int32(_KS1 + np.int32(3))
    x0, x1 = _rounds(x0, x1, _ROT2)
    x0 = x0 + _KS1
    x1 = x1 + np.int32(_KS2 + np.int32(4))
    x0, x1 = _rounds(x0, x1, _ROT1)
    x0 = x0 + _KS2
    x1 = x1 + np.int32(5)
    return x0 ^ x1


def _bits_to_u(bits):
    fbits = lax.shift_right_logical(bits, np.int32(9)) | _ONE_BITS
    return lax.bitcast_convert_type(fbits, jnp.float32) - np.float32(1.0) \
        + _TINY


def _update_running(best_val, best_idx, val, ci, base):
    bmax = jnp.max(val, axis=1, keepdims=True)
    bidx = base + jnp.min(jnp.where(val == bmax, ci, _BIG_IDX), axis=1,
                          keepdims=True)
    upd = bmax > best_val[...]
    best_val[...] = jnp.where(upd, bmax, best_val[...])
    best_idx[...] = jnp.where(upd, bidx, best_idx[...])


def _main_kernel(logits_ref, val_out, idx_out, best_val, best_idx, seed_ref,
                 *, blk, cols, limit, nblk):
    i = pl.program_id(0)
    rows = logits_ref.shape[0]

    @pl.when(i == 0)
    def _init():
        best_val[...] = jnp.full_like(best_val, _NEG_INF)
        best_idx[...] = jnp.zeros_like(best_idx)
        col0 = lax.broadcasted_iota(jnp.int32, (rows, blk), 1)
        row = lax.broadcasted_iota(jnp.int32, (rows, blk), 0)
        seed_ref[...] = row * np.int32(cols) + col0 + _KS1

    base = i * blk
    bits = _threefry_bits(seed_ref[...] + base)
    u = _bits_to_u(bits)
    g = -jnp.log(-jnp.log(u))
    val = g + logits_ref[...]
    ci = lax.broadcasted_iota(jnp.int32, (rows, blk), 1)
    val = jnp.where(ci < np.int32(limit) - base, val, _NEG_INF)
    _update_running(best_val, best_idx, val, ci, base)

    @pl.when(i == nblk - 1)
    def _emit():
        val_out[...] = best_val[...]
        idx_out[...] = best_idx[...]


def _tail_kernel(u_ref, logits_ref, pval_ref, pidx_ref, out_ref, best_val,
                 best_idx, *, blk, cols, start, nblk):
    i = pl.program_id(0)
    rows = u_ref.shape[0]

    @pl.when(i == 0)
    def _init():
        best_val[...] = pval_ref[...]
        best_idx[...] = pidx_ref[...]

    base = start + i * blk
    g = -jnp.log(-jnp.log(u_ref[...]))
    val = g + logits_ref[...]
    ci = lax.broadcasted_iota(jnp.int32, (rows, blk), 1)
    val = jnp.where(ci < np.int32(cols) - base, val, _NEG_INF)
    _update_running(best_val, best_idx, val, ci, base)

    @pl.when(i == nblk - 1)
    def _emit():
        out_ref[...] = best_idx[...]


def _sc_u_kernel(o_hbm, buf, sem, *, rows, cols, start, sc_cols):
    core = lax.axis_index("core")
    sub = lax.axis_index("subcore")
    sc_id = core * np.int32(16) + sub
    nchunk = sc_cols // _SC_CHUNK
    lane = lax.iota(jnp.int32, 16)

    @pl.loop(0, _SC_ROWS_PER_SUBCORE)
    def _row(rr):
        r = sc_id * np.int32(_SC_ROWS_PER_SUBCORE) + rr
        n_row = r * np.int32(cols) + np.int32(start + _KS1)

        @pl.loop(0, nchunk)
        def _chunk(ch):
            n_chunk = n_row + ch * np.int32(_SC_CHUNK)

            @pl.loop(0, _SC_CHUNK // 16)
            def _vec(v):
                x1 = (n_chunk + v * np.int32(16)) + lane
                buf[pl.ds(v * 16, 16)] = _bits_to_u(_threefry_bits(x1))

            pltpu.async_copy(
                buf, o_hbm.at[r, pl.ds(ch * _SC_CHUNK, _SC_CHUNK)],
                sem).wait()


def kernel(logits):
    rows, cols = logits.shape
    blk = _BLK
    sc_cols = 98304 if cols > 200000 else 0
    limit = cols - sc_cols
    nblk_main = (limit + blk - 1) // blk

    pval, pidx = pl.pallas_call(
        functools.partial(_main_kernel, blk=blk, cols=cols, limit=limit,
                          nblk=nblk_main),
        grid=(nblk_main,),
        in_specs=[pl.BlockSpec((rows, blk), lambda i: (0, i))],
        out_specs=[pl.BlockSpec((rows, 1), lambda i: (0, 0)),
                   pl.BlockSpec((rows, 1), lambda i: (0, 0))],
        out_shape=[jax.ShapeDtypeStruct((rows, 1), jnp.float32),
                   jax.ShapeDtypeStruct((rows, 1), jnp.int32)],
        scratch_shapes=[
            pltpu.VMEM((rows, 1), jnp.float32),
            pltpu.VMEM((rows, 1), jnp.int32),
            pltpu.VMEM((rows, blk), jnp.int32),
        ],
    )(logits)

    if sc_cols == 0:
        return pidx.reshape(-1)

    mesh = plsc.VectorSubcoreMesh(core_axis_name="core",
                                  subcore_axis_name="subcore")
    sc_fn = pl.kernel(
        functools.partial(_sc_u_kernel, rows=rows, cols=cols, start=limit,
                          sc_cols=sc_cols),
        out_type=jax.ShapeDtypeStruct((rows, sc_cols), jnp.float32),
        mesh=mesh,
        scratch_types=[pltpu.VMEM((_SC_CHUNK,), jnp.float32),
                       pltpu.SemaphoreType.DMA],
    )
    u_tail = sc_fn()

    nblk_tail = sc_cols // blk
    logits_tail = lax.slice(logits, (0, limit), (rows, cols))
    out = pl.pallas_call(
        functools.partial(_tail_kernel, blk=blk, cols=cols, start=limit,
                          nblk=nblk_tail),
        grid=(nblk_tail,),
        in_specs=[
            pl.BlockSpec((rows, blk), lambda i: (0, i)),
            pl.BlockSpec((rows, blk), lambda i: (0, i)),
            pl.BlockSpec((rows, 1), lambda i: (0, 0)),
            pl.BlockSpec((rows, 1), lambda i: (0, 0)),
        ],
        out_specs=pl.BlockSpec((rows, 1), lambda i: (0, 0)),
        out_shape=jax.ShapeDtypeStruct((rows, 1), jnp.int32),
        scratch_shapes=[
            pltpu.VMEM((rows, 1), jnp.float32),
            pltpu.VMEM((rows, 1), jnp.int32),
        ],
    )(u_tail, logits_tail, pval, pidx)
    return out.reshape(-1)


# speedup vs baseline: 1.3148x; 1.1680x over previous
import functools
import numpy as np

import jax
import jax.numpy as jnp
from jax import lax
from jax.experimental import pallas as pl
from jax.experimental.pallas import tpu as pltpu
from jax.experimental.pallas import tpu_sc as plsc

_ROT1 = (13, 15, 26, 6)
_ROT2 = (17, 29, 16, 24)
_KS1 = np.int32(42)
_KS2 = np.int32(np.uint32(0) ^ np.uint32(42) ^ np.uint32(0x1BD11BDA))
_TINY = np.float32(np.finfo(np.float32).tiny)
_ONE_BITS = np.int32(0x3F800000)
_NEG_INF = np.float32(-np.inf)
_BIG_IDX = np.int32(2**31 - 1)

_BLK = 2048
_SC_CHUNK = 2048
_SC_ROWS_PER_SUBCORE = 2


def _rotl(x, r):
    return lax.shift_left(x, np.int32(r)) | lax.shift_right_logical(
        x, np.int32(32 - r))


def _rounds(x0, x1, rots):
    for r in rots:
        x0 = x0 + x1
        x1 = _rotl(x1, r) ^ x0
    return x0, x1


def _threefry_bits(x1):
    x0 = x1
    x1 = _rotl(x1, _ROT1[0]) ^ x0
    for r in _ROT1[1:]:
        x0 = x0 + x1
        x1 = _rotl(x1, r) ^ x0
    x0 = x0 + _KS1
    x1 = x1 + np.int32(_KS2 + np.int32(1))
    x0, x1 = _rounds(x0, x1, _ROT2)
    x0 = x0 + _KS2
    x1 = x1 + np.int32(2)
    x0, x1 = _rounds(x0, x1, _ROT1)
    x1 = x1 + np.int32(_KS1 + np.int32(3))
    x0, x1 = _rounds(x0, x1, _ROT2)
    x0 = x0 + _KS1
    x1 = x1 + np.int32(_KS2 + np.int32(4))
    x0, x1 = _rounds(x0, x1, _ROT1)
    x0 = x0 + _KS2
    x1 = x1 + np.int32(5)
    return x0 ^ x1


def _bits_to_u(bits):
    fbits = lax.shift_right_logical(bits, np.int32(9)) | _ONE_BITS
    return lax.bitcast_convert_type(fbits, jnp.float32) - np.float32(1.0) \
        + _TINY


def _update_running(best_val, best_idx, val, ci, base):
    bmax = jnp.max(val, axis=1, keepdims=True)
    bidx = base + jnp.min(jnp.where(val == bmax, ci, _BIG_IDX), axis=1,
                          keepdims=True)
    upd = bmax > best_val[...]
    best_val[...] = jnp.where(upd, bmax, best_val[...])
    best_idx[...] = jnp.where(upd, bidx, best_idx[...])


def _main_kernel(logits_ref, val_out, idx_out, best_val, best_idx, seed_ref,
                 *, blk, cols, limit, nblk):
    i = pl.program_id(0)
    rows = logits_ref.shape[0]

    @pl.when(i == 0)
    def _init():
        best_val[...] = jnp.full_like(best_val, _NEG_INF)
        best_idx[...] = jnp.zeros_like(best_idx)
        col0 = lax.broadcasted_iota(jnp.int32, (rows, blk), 1)
        row = lax.broadcasted_iota(jnp.int32, (rows, blk), 0)
        seed_ref[...] = row * np.int32(cols) + col0 + _KS1

    base = i * blk
    bits = _threefry_bits(seed_ref[...] + base)
    u = _bits_to_u(bits)
    g = -jnp.log(-jnp.log(u))
    val = g + logits_ref[...]
    ci = lax.broadcasted_iota(jnp.int32, (rows, blk), 1)
    val = jnp.where(ci < np.int32(limit) - base, val, _NEG_INF)
    _update_running(best_val, best_idx, val, ci, base)

    @pl.when(i == nblk - 1)
    def _emit():
        val_out[...] = best_val[...]
        idx_out[...] = best_idx[...]


def _tail_kernel(u_ref, logits_ref, pval_ref, pidx_ref, out_ref, best_val,
                 best_idx, *, blk, cols, start, nblk):
    i = pl.program_id(0)
    rows = u_ref.shape[0]

    @pl.when(i == 0)
    def _init():
        best_val[...] = pval_ref[...]
        best_idx[...] = pidx_ref[...]

    base = start + i * blk
    g = -jnp.log(-jnp.log(u_ref[...]))
    val = g + logits_ref[...]
    ci = lax.broadcasted_iota(jnp.int32, (rows, blk), 1)
    val = jnp.where(ci < np.int32(cols) - base, val, _NEG_INF)
    _update_running(best_val, best_idx, val, ci, base)

    @pl.when(i == nblk - 1)
    def _emit():
        out_ref[...] = best_idx[...]


def _sc_u_kernel(o_hbm, buf, sem, *, rows, cols, start, sc_cols):
    core = lax.axis_index("core")
    sub = lax.axis_index("subcore")
    sc_id = core * np.int32(16) + sub
    nchunk = sc_cols // _SC_CHUNK
    lane = lax.iota(jnp.int32, 16)

    @pl.loop(0, _SC_ROWS_PER_SUBCORE)
    def _row(rr):
        r = sc_id * np.int32(_SC_ROWS_PER_SUBCORE) + rr
        n_row = r * np.int32(cols) + np.int32(start + _KS1)

        @pl.loop(0, nchunk)
        def _chunk(ch):
            n_chunk = n_row + ch * np.int32(_SC_CHUNK)

            @pl.loop(0, _SC_CHUNK // 16)
            def _vec(v):
                x1 = (n_chunk + v * np.int32(16)) + lane
                buf[pl.ds(v * 16, 16)] = _bits_to_u(_threefry_bits(x1))

            pltpu.async_copy(
                buf, o_hbm.at[r, pl.ds(ch * _SC_CHUNK, _SC_CHUNK)],
                sem).wait()


def kernel(logits):
    rows, cols = logits.shape
    blk = _BLK
    use_sc = rows == 64 and cols > 200000
    if use_sc:
        nblk_main = max(1, int(round(0.709 * cols / blk)))
        limit = nblk_main * blk
        sc_cols = ((cols - limit + _SC_CHUNK - 1) // _SC_CHUNK) * _SC_CHUNK
    else:
        limit = cols
        sc_cols = 0
        nblk_main = (limit + blk - 1) // blk

    pval, pidx = pl.pallas_call(
        functools.partial(_main_kernel, blk=blk, cols=cols, limit=limit,
                          nblk=nblk_main),
        grid=(nblk_main,),
        in_specs=[pl.BlockSpec((rows, blk), lambda i: (0, i))],
        out_specs=[pl.BlockSpec((rows, 1), lambda i: (0, 0)),
                   pl.BlockSpec((rows, 1), lambda i: (0, 0))],
        out_shape=[jax.ShapeDtypeStruct((rows, 1), jnp.float32),
                   jax.ShapeDtypeStruct((rows, 1), jnp.int32)],
        scratch_shapes=[
            pltpu.VMEM((rows, 1), jnp.float32),
            pltpu.VMEM((rows, 1), jnp.int32),
            pltpu.VMEM((rows, blk), jnp.int32),
        ],
    )(logits)

    if sc_cols == 0:
        return pidx.reshape(-1)

    mesh = plsc.VectorSubcoreMesh(core_axis_name="core",
                                  subcore_axis_name="subcore")
    sc_fn = pl.kernel(
        functools.partial(_sc_u_kernel, rows=rows, cols=cols, start=limit,
                          sc_cols=sc_cols),
        out_type=jax.ShapeDtypeStruct((rows, sc_cols), jnp.float32),
        mesh=mesh,
        scratch_types=[pltpu.VMEM((_SC_CHUNK,), jnp.float32),
                       pltpu.SemaphoreType.DMA],
    )
    u_tail = sc_fn()

    nblk_tail = sc_cols // blk
    nblk_off = limit // blk
    out = pl.pallas_call(
        functools.partial(_tail_kernel, blk=blk, cols=cols, start=limit,
                          nblk=nblk_tail),
        grid=(nblk_tail,),
        in_specs=[
            pl.BlockSpec((rows, blk), lambda i: (0, i)),
            pl.BlockSpec((rows, blk), lambda i, o=nblk_off: (0, o + i)),
            pl.BlockSpec((rows, 1), lambda i: (0, 0)),
            pl.BlockSpec((rows, 1), lambda i: (0, 0)),
        ],
        out_specs=pl.BlockSpec((rows, 1), lambda i: (0, 0)),
        out_shape=jax.ShapeDtypeStruct((rows, 1), jnp.int32),
        scratch_shapes=[
            pltpu.VMEM((rows, 1), jnp.float32),
            pltpu.VMEM((rows, 1), jnp.int32),
        ],
    )(u_tail, logits, pval, pidx)
    return out.reshape(-1)


# speedup vs baseline: 1.3168x; 1.0015x over previous
import functools
import numpy as np

import jax
import jax.numpy as jnp
from jax import lax
from jax.experimental import pallas as pl
from jax.experimental.pallas import tpu as pltpu
from jax.experimental.pallas import tpu_sc as plsc

_ROT1 = (13, 15, 26, 6)
_ROT2 = (17, 29, 16, 24)
_KS1 = np.int32(42)
_KS2 = np.int32(np.uint32(0) ^ np.uint32(42) ^ np.uint32(0x1BD11BDA))
_TINY = np.float32(np.finfo(np.float32).tiny)
_ONE_BITS = np.int32(0x3F800000)
_NEG_INF = np.float32(-np.inf)
_BIG_IDX = np.int32(2**31 - 1)

_BLK = 2048
_SC_CHUNK = 2048
_SC_ROWS_PER_SUBCORE = 2


def _rotl(x, r):
    return lax.shift_left(x, np.int32(r)) | lax.shift_right_logical(
        x, np.int32(32 - r))


def _rounds(x0, x1, rots):
    for r in rots:
        x0 = x0 + x1
        x1 = _rotl(x1, r) ^ x0
    return x0, x1


def _threefry_bits(x1):
    x0 = x1
    x1 = _rotl(x1, _ROT1[0]) ^ x0
    for r in _ROT1[1:]:
        x0 = x0 + x1
        x1 = _rotl(x1, r) ^ x0
    x0 = x0 + _KS1
    x1 = x1 + np.int32(_KS2 + np.int32(1))
    x0, x1 = _rounds(x0, x1, _ROT2)
    x0 = x0 + _KS2
    x1 = x1 + np.int32(2)
    x0, x1 = _rounds(x0, x1, _ROT1)
    x1 = x1 + np.int32(_KS1 + np.int32(3))
    x0, x1 = _rounds(x0, x1, _ROT2)
    x0 = x0 + _KS1
    x1 = x1 + np.int32(_KS2 + np.int32(4))
    x0, x1 = _rounds(x0, x1, _ROT1)
    x0 = x0 + _KS2
    x1 = x1 + np.int32(5)
    return x0 ^ x1


def _bits_to_u(bits):
    fbits = lax.shift_right_logical(bits, np.int32(9)) | _ONE_BITS
    return lax.bitcast_convert_type(fbits, jnp.float32) - np.float32(1.0) \
        + _TINY


def _update_running(best_val, best_idx, val, ci, base):
    bmax = jnp.max(val, axis=1, keepdims=True)
    bidx = base + jnp.min(jnp.where(val == bmax, ci, _BIG_IDX), axis=1,
                          keepdims=True)
    upd = bmax > best_val[...]
    best_val[...] = jnp.where(upd, bmax, best_val[...])
    best_idx[...] = jnp.where(upd, bidx, best_idx[...])


def _main_kernel(logits_ref, val_out, idx_out, best_val, best_idx, seed_ref,
                 *, blk, cols, limit, nblk):
    i = pl.program_id(0)
    rows = logits_ref.shape[0]

    @pl.when(i == 0)
    def _init():
        best_val[...] = jnp.full_like(best_val, _NEG_INF)
        best_idx[...] = jnp.zeros_like(best_idx)
        col0 = lax.broadcasted_iota(jnp.int32, (rows, blk), 1)
        row = lax.broadcasted_iota(jnp.int32, (rows, blk), 0)
        seed_ref[...] = row * np.int32(cols) + col0 + _KS1

    base = i * blk
    bits = _threefry_bits(seed_ref[...] + base)
    u = _bits_to_u(bits)
    g = -jnp.log(-jnp.log(u))
    val = g + logits_ref[...]
    ci = lax.broadcasted_iota(jnp.int32, (rows, blk), 1)
    val = jnp.where(ci < np.int32(limit) - base, val, _NEG_INF)
    _update_running(best_val, best_idx, val, ci, base)

    @pl.when(i == nblk - 1)
    def _emit():
        val_out[...] = best_val[...]
        idx_out[...] = best_idx[...]


def _tail_kernel(u_ref, logits_ref, pval_ref, pidx_ref, out_ref, best_val,
                 best_idx, *, blk, cols, start, nblk):
    i = pl.program_id(0)
    rows = u_ref.shape[0]

    @pl.when(i == 0)
    def _init():
        best_val[...] = pval_ref[...]
        best_idx[...] = pidx_ref[...]

    base = start + i * blk
    g = -jnp.log(-jnp.log(u_ref[...]))
    val = g + logits_ref[...]
    ci = lax.broadcasted_iota(jnp.int32, (rows, blk), 1)
    val = jnp.where(ci < np.int32(cols) - base, val, _NEG_INF)
    _update_running(best_val, best_idx, val, ci, base)

    @pl.when(i == nblk - 1)
    def _emit():
        out_ref[...] = best_idx[...]


def _sc_u_kernel(o_hbm, buf, sem, *, rows, cols, start, sc_cols):
    core = lax.axis_index("core")
    sub = lax.axis_index("subcore")
    sc_id = core * np.int32(16) + sub
    nchunk = sc_cols // _SC_CHUNK
    lane = lax.iota(jnp.int32, 16)

    @pl.loop(0, _SC_ROWS_PER_SUBCORE)
    def _row(rr):
        r = sc_id * np.int32(_SC_ROWS_PER_SUBCORE) + rr
        n_row = r * np.int32(cols) + np.int32(start + _KS1)

        @pl.loop(0, nchunk)
        def _chunk(ch):
            p = lax.rem(ch, np.int32(2))

            @pl.when(ch >= 2)
            def _reclaim():
                pltpu.make_async_copy(
                    buf.at[0],
                    o_hbm.at[r, pl.ds(0, _SC_CHUNK)], sem).wait()

            n_chunk = n_row + ch * np.int32(_SC_CHUNK)

            @pl.loop(0, _SC_CHUNK // 16)
            def _vec(v):
                x1 = (n_chunk + v * np.int32(16)) + lane
                buf[p, pl.ds(v * 16, 16)] = _bits_to_u(_threefry_bits(x1))

            pltpu.async_copy(
                buf.at[p], o_hbm.at[r, pl.ds(ch * _SC_CHUNK, _SC_CHUNK)],
                sem)

        @pl.when(nchunk >= 1)
        def _drain0():
            pltpu.make_async_copy(
                buf.at[0], o_hbm.at[r, pl.ds(0, _SC_CHUNK)], sem).wait()

        @pl.when(nchunk >= 2)
        def _drain1():
            pltpu.make_async_copy(
                buf.at[0], o_hbm.at[r, pl.ds(0, _SC_CHUNK)], sem).wait()


def kernel(logits):
    rows, cols = logits.shape
    blk = _BLK
    use_sc = rows == 64 and cols > 200000
    if use_sc:
        nblk_main = max(1, int(round(0.709 * cols / blk)))
        limit = nblk_main * blk
        sc_cols = ((cols - limit + _SC_CHUNK - 1) // _SC_CHUNK) * _SC_CHUNK
    else:
        limit = cols
        sc_cols = 0
        nblk_main = (limit + blk - 1) // blk

    pval, pidx = pl.pallas_call(
        functools.partial(_main_kernel, blk=blk, cols=cols, limit=limit,
                          nblk=nblk_main),
        grid=(nblk_main,),
        in_specs=[pl.BlockSpec((rows, blk), lambda i: (0, i))],
        out_specs=[pl.BlockSpec((rows, 1), lambda i: (0, 0)),
                   pl.BlockSpec((rows, 1), lambda i: (0, 0))],
        out_shape=[jax.ShapeDtypeStruct((rows, 1), jnp.float32),
                   jax.ShapeDtypeStruct((rows, 1), jnp.int32)],
        scratch_shapes=[
            pltpu.VMEM((rows, 1), jnp.float32),
            pltpu.VMEM((rows, 1), jnp.int32),
            pltpu.VMEM((rows, blk), jnp.int32),
        ],
    )(logits)

    if sc_cols == 0:
        return pidx.reshape(-1)

    mesh = plsc.VectorSubcoreMesh(core_axis_name="core",
                                  subcore_axis_name="subcore")
    sc_fn = pl.kernel(
        functools.partial(_sc_u_kernel, rows=rows, cols=cols, start=limit,
                          sc_cols=sc_cols),
        out_type=jax.ShapeDtypeStruct((rows, sc_cols), jnp.float32),
        mesh=mesh,
        scratch_types=[pltpu.VMEM((2, _SC_CHUNK), jnp.float32),
                       pltpu.SemaphoreType.DMA],
    )
    u_tail = sc_fn()

    nblk_tail = sc_cols // blk
    nblk_off = limit // blk
    out = pl.pallas_call(
        functools.partial(_tail_kernel, blk=blk, cols=cols, start=limit,
                          nblk=nblk_tail),
        grid=(nblk_tail,),
        in_specs=[
            pl.BlockSpec((rows, blk), lambda i: (0, i)),
            pl.BlockSpec((rows, blk), lambda i, o=nblk_off: (0, o + i)),
            pl.BlockSpec((rows, 1), lambda i: (0, 0)),
            pl.BlockSpec((rows, 1), lambda i: (0, 0)),
        ],
        out_specs=pl.BlockSpec((rows, 1), lambda i: (0, 0)),
        out_shape=jax.ShapeDtypeStruct((rows, 1), jnp.int32),
        scratch_shapes=[
            pltpu.VMEM((rows, 1), jnp.float32),
            pltpu.VMEM((rows, 1), jnp.int32),
        ],
    )(u_tail, logits, pval, pidx)
    return out.reshape(-1)
